# baseline (speedup 1.0000x reference)
import functools

import jax
import jax.numpy as jnp
from jax import lax
from jax.experimental import pallas as pl
from jax.experimental.pallas import tpu as pltpu
from jax.experimental.pallas import tpu_sc as plsc

VOCAB = 1000000
EMBED_DIM = 64
NUM_CLASSES = 1000
BATCH = 16384
SEQLEN = 50

_PAD_CLASSES = 1024
_NC = 2
_NS = 16
_NW = _NC * _NS
_ROWS_PER_W = BATCH // _NW
_PAIRS_PER_W = _ROWS_PER_W // 2
_IDX_PER_GATHER = 2 * SEQLEN
_QV = EMBED_DIM // 16


_NBUF = 4


def _sc_pool_body(x_hbm, tab_hbm, out_hbm, idx_v, b0, b1, b2, b3, out_v,
                  s0, s1, s2, s3):
    wid = lax.axis_index("s") * _NC + lax.axis_index("c")
    pair_base = wid * _PAIRS_PER_W

    pltpu.sync_copy(x_hbm.at[pl.ds(pair_base, _PAIRS_PER_W), :], idx_v)

    bufs = (b0, b1, b2, b3)
    sems = (s0, s1, s2, s3)

    for j in range(_NBUF - 1):
        pltpu.async_copy(tab_hbm.at[idx_v.at[j]], bufs[j], sems[j])

    def accumulate(buf, j):
        acc = [jnp.zeros((16,), jnp.float32) for _ in range(2 * _QV)]
        for r in range(SEQLEN):
            for half in range(2):
                for q in range(_QV):
                    acc[half * _QV + q] = acc[half * _QV + q] + buf[
                        half * SEQLEN + r, pl.ds(q * 16, 16)]
        scale = jnp.float32(1.0 / SEQLEN)
        for half in range(2):
            for q in range(_QV):
                out_v[2 * j + half, pl.ds(q * 16, 16)] = acc[half * _QV + q] * scale

    @pl.loop(0, _PAIRS_PER_W, step=_NBUF)
    def _(g):
        for b in range(_NBUF):
            j = g + b
            nxt = (b + _NBUF - 1) % _NBUF

            @pl.when(j + _NBUF - 1 < _PAIRS_PER_W)
            def _():
                pltpu.async_copy(
                    tab_hbm.at[idx_v.at[j + _NBUF - 1]], bufs[nxt], sems[nxt])

            pltpu.make_async_copy(tab_hbm.at[idx_v.at[j]], bufs[b], sems[b]).wait()
            accumulate(bufs[b], j)

    pltpu.sync_copy(out_v, out_hbm.at[pl.ds(wid * _ROWS_PER_W, _ROWS_PER_W), :])


def _make_sc_pool():
    mesh = plsc.VectorSubcoreMesh(core_axis_name="c", subcore_axis_name="s")
    return pl.kernel(
        _sc_pool_body,
        out_type=jax.ShapeDtypeStruct((BATCH, EMBED_DIM), jnp.float32),
        mesh=mesh,
        compiler_params=pltpu.CompilerParams(use_tc_tiling_on_sc=False),
        scratch_types=[
            pltpu.VMEM((_PAIRS_PER_W, _IDX_PER_GATHER), jnp.int32),
            pltpu.VMEM((_IDX_PER_GATHER, EMBED_DIM), jnp.float32),
            pltpu.VMEM((_IDX_PER_GATHER, EMBED_DIM), jnp.float32),
            pltpu.VMEM((_IDX_PER_GATHER, EMBED_DIM), jnp.float32),
            pltpu.VMEM((_IDX_PER_GATHER, EMBED_DIM), jnp.float32),
            pltpu.VMEM((_ROWS_PER_W, EMBED_DIM), jnp.float32),
            pltpu.SemaphoreType.DMA,
            pltpu.SemaphoreType.DMA,
            pltpu.SemaphoreType.DMA,
            pltpu.SemaphoreType.DMA,
        ],
    )


_BM = 512


def _tc_head_body(x_ref, wt_ref, b_ref, o_ref):
    logits = (
        jnp.dot(x_ref[...], wt_ref[...], preferred_element_type=jnp.float32)
        + b_ref[...]
    )
    m = jnp.max(logits, axis=-1, keepdims=True)
    e = jnp.exp(logits - m)
    o_ref[...] = e / jnp.sum(e, axis=-1, keepdims=True)


@functools.partial(jax.jit, static_argnames=())
def _run(x, emb_table, W, b):
    x_pairs = jnp.reshape(x.astype(jnp.int32), (BATCH // 2, _IDX_PER_GATHER))
    pooled = _make_sc_pool()(x_pairs, emb_table)

    wt = jnp.zeros((EMBED_DIM, _PAD_CLASSES), jnp.float32)
    wt = wt.at[:, :NUM_CLASSES].set(W.T)
    bp = jnp.full((1, _PAD_CLASSES), -1e30, jnp.float32)
    bp = bp.at[0, :NUM_CLASSES].set(b)

    out = pl.pallas_call(
        _tc_head_body,
        grid=(BATCH // _BM,),
        in_specs=[
            pl.BlockSpec((_BM, EMBED_DIM), lambda i: (i, 0)),
            pl.BlockSpec((EMBED_DIM, _PAD_CLASSES), lambda i: (0, 0)),
            pl.BlockSpec((1, _PAD_CLASSES), lambda i: (0, 0)),
        ],
        out_specs=pl.BlockSpec((_BM, _PAD_CLASSES), lambda i: (i, 0)),
        out_shape=jax.ShapeDtypeStruct((BATCH, _PAD_CLASSES), jnp.float32),
    )(pooled, wt, bp)
    return out[:, :NUM_CLASSES]


def kernel(x, emb_table, W, b):
    return _run(x, emb_table, W, b)


# speedup vs baseline: 1.2214x; 1.2214x over previous
import functools

import jax
import jax.numpy as jnp
from jax import lax
from jax.experimental import pallas as pl
from jax.experimental.pallas import tpu as pltpu
from jax.experimental.pallas import tpu_sc as plsc

VOCAB = 1000000
EMBED_DIM = 64
NUM_CLASSES = 1000
BATCH = 16384
SEQLEN = 50

_PAD_CLASSES = 1024
_NC = 2
_NS = 16
_NW = _NC * _NS
_ROWS_PER_W = BATCH // _NW
_PAIRS_PER_W = _ROWS_PER_W // 2
_IDX_PER_GATHER = 2 * SEQLEN
_QV = EMBED_DIM // 16


_NBUF = 4


def _sc_pool_body(x_hbm, tab_hbm, out_hbm, idx_v, b0, b1, b2, b3, out_v,
                  s0, s1, s2, s3):
    wid = lax.axis_index("s") * _NC + lax.axis_index("c")
    pair_base = wid * _PAIRS_PER_W

    pltpu.sync_copy(x_hbm.at[pl.ds(pair_base, _PAIRS_PER_W), :], idx_v)

    bufs = (b0, b1, b2, b3)
    sems = (s0, s1, s2, s3)

    for j in range(_NBUF - 1):
        pltpu.async_copy(tab_hbm.at[idx_v.at[j]], bufs[j], sems[j])

    _U = 5

    def accumulate(buf, j):
        def rbody(t, acc):
            acc = list(acc)
            for u in range(_U):
                r = t * _U + u
                for half in range(2):
                    for q in range(_QV):
                        acc[half * _QV + q] = acc[half * _QV + q] + buf[
                            half * SEQLEN + r, pl.ds(q * 16, 16)]
            return tuple(acc)

        zeros = tuple(jnp.zeros((16,), jnp.float32) for _ in range(2 * _QV))
        acc = lax.fori_loop(0, SEQLEN // _U, rbody, zeros)
        scale = jnp.float32(1.0 / SEQLEN)
        for half in range(2):
            for q in range(_QV):
                out_v[2 * j + half, pl.ds(q * 16, 16)] = acc[half * _QV + q] * scale

    @pl.loop(0, _PAIRS_PER_W, step=_NBUF)
    def _(g):
        for b in range(_NBUF):
            j = g + b
            nxt = (b + _NBUF - 1) % _NBUF

            @pl.when(j + _NBUF - 1 < _PAIRS_PER_W)
            def _():
                pltpu.async_copy(
                    tab_hbm.at[idx_v.at[j + _NBUF - 1]], bufs[nxt], sems[nxt])

            pltpu.make_async_copy(tab_hbm.at[idx_v.at[j]], bufs[b], sems[b]).wait()
            accumulate(bufs[b], j)

    pltpu.sync_copy(out_v, out_hbm.at[pl.ds(wid * _ROWS_PER_W, _ROWS_PER_W), :])


def _make_sc_pool():
    mesh = plsc.VectorSubcoreMesh(core_axis_name="c", subcore_axis_name="s")
    return pl.kernel(
        _sc_pool_body,
        out_type=jax.ShapeDtypeStruct((BATCH, EMBED_DIM), jnp.float32),
        mesh=mesh,
        compiler_params=pltpu.CompilerParams(use_tc_tiling_on_sc=False),
        scratch_types=[
            pltpu.VMEM((_PAIRS_PER_W, _IDX_PER_GATHER), jnp.int32),
            pltpu.VMEM((_IDX_PER_GATHER, EMBED_DIM), jnp.float32),
            pltpu.VMEM((_IDX_PER_GATHER, EMBED_DIM), jnp.float32),
            pltpu.VMEM((_IDX_PER_GATHER, EMBED_DIM), jnp.float32),
            pltpu.VMEM((_IDX_PER_GATHER, EMBED_DIM), jnp.float32),
            pltpu.VMEM((_ROWS_PER_W, EMBED_DIM), jnp.float32),
            pltpu.SemaphoreType.DMA,
            pltpu.SemaphoreType.DMA,
            pltpu.SemaphoreType.DMA,
            pltpu.SemaphoreType.DMA,
        ],
    )


_BM = 512


def _tc_head_body(x_ref, wt_ref, b_ref, o_ref):
    logits = (
        jnp.dot(x_ref[...], wt_ref[...], preferred_element_type=jnp.float32)
        + b_ref[...]
    )
    m = jnp.max(logits, axis=-1, keepdims=True)
    e = jnp.exp(logits - m)
    o_ref[...] = e / jnp.sum(e, axis=-1, keepdims=True)


@functools.partial(jax.jit, static_argnames=())
def _run(x, emb_table, W, b):
    x_pairs = jnp.reshape(x.astype(jnp.int32), (BATCH // 2, _IDX_PER_GATHER))
    pooled = _make_sc_pool()(x_pairs, emb_table)

    wt = jnp.zeros((EMBED_DIM, _PAD_CLASSES), jnp.float32)
    wt = wt.at[:, :NUM_CLASSES].set(W.T)
    bp = jnp.full((1, _PAD_CLASSES), -1e30, jnp.float32)
    bp = bp.at[0, :NUM_CLASSES].set(b)

    out = pl.pallas_call(
        _tc_head_body,
        grid=(BATCH // _BM,),
        in_specs=[
            pl.BlockSpec((_BM, EMBED_DIM), lambda i: (i, 0)),
            pl.BlockSpec((EMBED_DIM, _PAD_CLASSES), lambda i: (0, 0)),
            pl.BlockSpec((1, _PAD_CLASSES), lambda i: (0, 0)),
        ],
        out_specs=pl.BlockSpec((_BM, _PAD_CLASSES), lambda i: (i, 0)),
        out_shape=jax.ShapeDtypeStruct((BATCH, _PAD_CLASSES), jnp.float32),
    )(pooled, wt, bp)
    return out[:, :NUM_CLASSES]


def kernel(x, emb_table, W, b):
    return _run(x, emb_table, W, b)


# speedup vs baseline: 1.2554x; 1.0278x over previous
import functools

import jax
import jax.numpy as jnp
from jax import lax
from jax.experimental import pallas as pl
from jax.experimental.pallas import tpu as pltpu
from jax.experimental.pallas import tpu_sc as plsc

VOCAB = 1000000
EMBED_DIM = 64
NUM_CLASSES = 1000
BATCH = 16384
SEQLEN = 50

_NC = 2
_NS = 16
_NW = _NC * _NS
_ROWS_PER_W = BATCH // _NW
_PAIRS_PER_W = _ROWS_PER_W // 2
_IDX_PER_GATHER = 2 * SEQLEN
_IDX_PITCH = 104
_QV = EMBED_DIM // 16
_NBUF = 4

_BC = 2048
_N_CBLK = (VOCAB + _BC - 1) // _BC
_TAB_ROWS = _N_CBLK * _BC


def _tc_repack_body(in_ref, o_ref):
    o_ref[:, 0:EMBED_DIM] = jnp.transpose(in_ref[...], (1, 0))


def _sc_pool_body(x_hbm, tab_hbm, out_hbm, idx_v, b0, b1, b2, b3, out_v,
                  s0, s1, s2, s3):
    wid = lax.axis_index("s") * _NC + lax.axis_index("c")
    n_idx = _PAIRS_PER_W * _IDX_PITCH

    pltpu.sync_copy(x_hbm.at[pl.ds(wid * n_idx, n_idx)], idx_v)

    bufs = (b0, b1, b2, b3)
    sems = (s0, s1, s2, s3)

    def start(j, b):
        pltpu.async_copy(
            tab_hbm.at[idx_v.at[pl.ds(j * _IDX_PITCH, _IDX_PER_GATHER)]],
            bufs[b], sems[b])

    def wait(j, b):
        pltpu.make_async_copy(
            tab_hbm.at[idx_v.at[pl.ds(j * _IDX_PITCH, _IDX_PER_GATHER)]],
            bufs[b], sems[b]).wait()

    for j in range(_NBUF - 1):
        start(j, j)

    _U = 5

    def accumulate(buf, j):
        def rbody(t, acc):
            acc = list(acc)
            for u in range(_U):
                r = t * _U + u
                for half in range(2):
                    for q in range(_QV):
                        acc[half * _QV + q] = acc[half * _QV + q] + buf[
                            half * SEQLEN + r, pl.ds(q * 16, 16)]
            return tuple(acc)

        zeros = tuple(jnp.zeros((16,), jnp.float32) for _ in range(2 * _QV))
        acc = lax.fori_loop(0, SEQLEN // _U, rbody, zeros)
        scale = jnp.float32(1.0 / SEQLEN)
        for half in range(2):
            for q in range(_QV):
                out_v[pl.ds((2 * j + half) * EMBED_DIM + q * 16, 16)] = (
                    acc[half * _QV + q] * scale)

    @pl.loop(0, _PAIRS_PER_W, step=_NBUF)
    def _(g):
        for b in range(_NBUF):
            j = g + b
            nxt = (b + _NBUF - 1) % _NBUF

            @pl.when(j + _NBUF - 1 < _PAIRS_PER_W)
            def _():
                start(j + _NBUF - 1, nxt)

            wait(j, b)
            accumulate(bufs[b], j)

    n_out = _ROWS_PER_W * EMBED_DIM
    pltpu.sync_copy(out_v, out_hbm.at[pl.ds(wid * n_out, n_out)])


def _make_sc_pool():
    mesh = plsc.VectorSubcoreMesh(core_axis_name="c", subcore_axis_name="s")
    return pl.kernel(
        _sc_pool_body,
        out_type=jax.ShapeDtypeStruct((BATCH * EMBED_DIM,), jnp.float32),
        mesh=mesh,
        compiler_params=pltpu.CompilerParams(use_tc_tiling_on_sc=True),
        scratch_types=[
            pltpu.VMEM((_PAIRS_PER_W * _IDX_PITCH,), jnp.int32),
            pltpu.VMEM((_IDX_PER_GATHER, 128), jnp.float32),
            pltpu.VMEM((_IDX_PER_GATHER, 128), jnp.float32),
            pltpu.VMEM((_IDX_PER_GATHER, 128), jnp.float32),
            pltpu.VMEM((_IDX_PER_GATHER, 128), jnp.float32),
            pltpu.VMEM((_ROWS_PER_W * EMBED_DIM,), jnp.float32),
            pltpu.SemaphoreType.DMA,
            pltpu.SemaphoreType.DMA,
            pltpu.SemaphoreType.DMA,
            pltpu.SemaphoreType.DMA,
        ],
    )


_BM = 512


def _tc_head_body(x_ref, w_ref, b_ref, o_ref):
    logits = (
        lax.dot_general(
            x_ref[...], w_ref[...], (((1,), (1,)), ((), ())),
            preferred_element_type=jnp.float32,
        )
        + b_ref[...]
    )
    m = jnp.max(logits, axis=-1, keepdims=True)
    e = jnp.exp(logits - m)
    o_ref[...] = e / jnp.sum(e, axis=-1, keepdims=True)


@functools.partial(jax.jit, static_argnames=())
def _run(x, emb_table, W, b):
    xg = jnp.reshape(x.astype(jnp.int32), (BATCH // 2, _IDX_PER_GATHER))
    xg = jnp.pad(xg, ((0, 0), (0, _IDX_PITCH - _IDX_PER_GATHER)))
    x_flat = jnp.reshape(xg, (-1,))

    tab = pl.pallas_call(
        _tc_repack_body,
        grid=(_N_CBLK,),
        in_specs=[pl.BlockSpec((EMBED_DIM, _BC), lambda i: (0, i))],
        out_specs=pl.BlockSpec((_BC, 128), lambda i: (i, 0)),
        out_shape=jax.ShapeDtypeStruct((_TAB_ROWS, 128), jnp.float32),
    )(emb_table.T)

    pooled = jnp.reshape(
        _make_sc_pool()(x_flat, tab), (BATCH, EMBED_DIM))

    out = pl.pallas_call(
        _tc_head_body,
        grid=(BATCH // _BM,),
        in_specs=[
            pl.BlockSpec((_BM, EMBED_DIM), lambda i: (i, 0)),
            pl.BlockSpec((NUM_CLASSES, EMBED_DIM), lambda i: (0, 0)),
            pl.BlockSpec((1, NUM_CLASSES), lambda i: (0, 0)),
        ],
        out_specs=pl.BlockSpec((_BM, NUM_CLASSES), lambda i: (i, 0)),
        out_shape=jax.ShapeDtypeStruct((BATCH, NUM_CLASSES), jnp.float32),
    )(pooled, W, jnp.reshape(b, (1, NUM_CLASSES)))
    return out


def kernel(x, emb_table, W, b):
    return _run(x, emb_table, W, b)


# speedup vs baseline: 1.8604x; 1.4819x over previous
import functools

import jax
import jax.numpy as jnp
from jax import lax
from jax.experimental import pallas as pl
from jax.experimental.pallas import tpu as pltpu
from jax.experimental.pallas import tpu_sc as plsc

VOCAB = 1000000
EMBED_DIM = 64
NUM_CLASSES = 1000
BATCH = 16384
SEQLEN = 50

_NC = 2
_NS = 16
_NW = _NC * _NS
_ROWS_PER_W = BATCH // _NW
_PAIRS_PER_W = _ROWS_PER_W // 2
_IDX_PER_GATHER = 2 * SEQLEN
_IDX_PITCH = 104
_QV = EMBED_DIM // 16
_NBUF = 4

_BC = 8192
_N_CBLK = (VOCAB + _BC - 1) // _BC
_TAB_ROWS = _N_CBLK * _BC


def _tc_repack_body(in_ref, o_ref):
    o_ref[:, 0:EMBED_DIM] = jnp.transpose(in_ref[...], (1, 0))


def _sc_pool_body(x_hbm, tab_hbm, out_hbm, idx_v, b0, b1, b2, b3, out_v,
                  s0, s1, s2, s3):
    wid = lax.axis_index("s") * _NC + lax.axis_index("c")
    n_idx = _PAIRS_PER_W * _IDX_PITCH

    pltpu.sync_copy(x_hbm.at[pl.ds(wid * n_idx, n_idx)], idx_v)

    bufs = (b0, b1, b2, b3)
    sems = (s0, s1, s2, s3)

    def start(j, b):
        pltpu.async_copy(
            tab_hbm.at[idx_v.at[pl.ds(j * _IDX_PITCH, _IDX_PER_GATHER)]],
            bufs[b], sems[b])

    def wait(j, b):
        pltpu.make_async_copy(
            tab_hbm.at[idx_v.at[pl.ds(j * _IDX_PITCH, _IDX_PER_GATHER)]],
            bufs[b], sems[b]).wait()

    for j in range(_NBUF - 1):
        start(j, j)

    _U = 5

    def accumulate(buf, j):
        def rbody(t, acc):
            acc = list(acc)
            for u in range(_U):
                r = t * _U + u
                for half in range(2):
                    for q in range(_QV):
                        acc[half * _QV + q] = acc[half * _QV + q] + buf[
                            half * SEQLEN + r, pl.ds(q * 16, 16)]
            return tuple(acc)

        zeros = tuple(jnp.zeros((16,), jnp.float32) for _ in range(2 * _QV))
        acc = lax.fori_loop(0, SEQLEN // _U, rbody, zeros)
        scale = jnp.float32(1.0 / SEQLEN)
        for half in range(2):
            for q in range(_QV):
                out_v[pl.ds((2 * j + half) * EMBED_DIM + q * 16, 16)] = (
                    acc[half * _QV + q] * scale)

    @pl.loop(0, _PAIRS_PER_W, step=_NBUF)
    def _(g):
        for b in range(_NBUF):
            j = g + b
            nxt = (b + _NBUF - 1) % _NBUF

            @pl.when(j + _NBUF - 1 < _PAIRS_PER_W)
            def _():
                start(j + _NBUF - 1, nxt)

            wait(j, b)
            accumulate(bufs[b], j)

    n_out = _ROWS_PER_W * EMBED_DIM
    pltpu.sync_copy(out_v, out_hbm.at[pl.ds(wid * n_out, n_out)])


def _make_sc_pool():
    mesh = plsc.VectorSubcoreMesh(core_axis_name="c", subcore_axis_name="s")
    return pl.kernel(
        _sc_pool_body,
        out_type=jax.ShapeDtypeStruct((BATCH * EMBED_DIM,), jnp.float32),
        mesh=mesh,
        compiler_params=pltpu.CompilerParams(use_tc_tiling_on_sc=True),
        scratch_types=[
            pltpu.VMEM((_PAIRS_PER_W * _IDX_PITCH,), jnp.int32),
            pltpu.VMEM((_IDX_PER_GATHER, 128), jnp.float32),
            pltpu.VMEM((_IDX_PER_GATHER, 128), jnp.float32),
            pltpu.VMEM((_IDX_PER_GATHER, 128), jnp.float32),
            pltpu.VMEM((_IDX_PER_GATHER, 128), jnp.float32),
            pltpu.VMEM((_ROWS_PER_W * EMBED_DIM,), jnp.float32),
            pltpu.SemaphoreType.DMA,
            pltpu.SemaphoreType.DMA,
            pltpu.SemaphoreType.DMA,
            pltpu.SemaphoreType.DMA,
        ],
    )


_BM = 512


def _tc_head_body(x_ref, w_ref, b_ref, o_ref):
    logits = (
        lax.dot_general(
            w_ref[...], x_ref[...], (((1,), (1,)), ((), ())),
            preferred_element_type=jnp.float32,
        )
        + b_ref[...]
    )
    m = jnp.max(logits, axis=0, keepdims=True)
    e = jnp.exp(logits - m)
    o_ref[...] = e / jnp.sum(e, axis=0, keepdims=True)


@functools.partial(jax.jit, static_argnames=())
def _run(x, emb_table, W, b):
    xg = jnp.reshape(x.astype(jnp.int32), (BATCH // 2, _IDX_PER_GATHER))
    xg = jnp.pad(xg, ((0, 0), (0, _IDX_PITCH - _IDX_PER_GATHER)))
    x_flat = jnp.reshape(xg, (-1,))

    tab = pl.pallas_call(
        _tc_repack_body,
        grid=(_N_CBLK,),
        in_specs=[pl.BlockSpec((EMBED_DIM, _BC), lambda i: (0, i))],
        out_specs=pl.BlockSpec((_BC, 128), lambda i: (i, 0)),
        out_shape=jax.ShapeDtypeStruct((_TAB_ROWS, 128), jnp.float32),
    )(emb_table.T)

    pooled = jnp.reshape(
        _make_sc_pool()(x_flat, tab), (BATCH, EMBED_DIM))

    out_t = pl.pallas_call(
        _tc_head_body,
        grid=(BATCH // _BM,),
        in_specs=[
            pl.BlockSpec((_BM, EMBED_DIM), lambda i: (i, 0)),
            pl.BlockSpec((NUM_CLASSES, EMBED_DIM), lambda i: (0, 0)),
            pl.BlockSpec((NUM_CLASSES, 1), lambda i: (0, 0)),
        ],
        out_specs=pl.BlockSpec((NUM_CLASSES, _BM), lambda i: (0, i)),
        out_shape=jax.ShapeDtypeStruct((NUM_CLASSES, BATCH), jnp.float32),
    )(pooled, W, jnp.reshape(b, (NUM_CLASSES, 1)))
    return jnp.transpose(out_t)


def kernel(x, emb_table, W, b):
    return _run(x, emb_table, W, b)


# speedup vs baseline: 1.9662x; 1.0569x over previous
import functools

import jax
import jax.numpy as jnp
from jax import lax
from jax.experimental import pallas as pl
from jax.experimental.pallas import tpu as pltpu
from jax.experimental.pallas import tpu_sc as plsc

VOCAB = 1000000
EMBED_DIM = 64
NUM_CLASSES = 1000
BATCH = 16384
SEQLEN = 50

_NC = 2
_NS = 16
_NW = _NC * _NS
_ROWS_PER_W = BATCH // _NW
_PAIRS_PER_W = _ROWS_PER_W // 2
_IDX_PER_GATHER = 2 * SEQLEN
_IDX_PITCH = 104
_QV = EMBED_DIM // 16
_NBUF = 4

_BC = 16384
_N_CBLK = (VOCAB + _BC - 1) // _BC
_TAB_ROWS = _N_CBLK * _BC


def _tc_repack_body(in_ref, o_ref):
    o_ref[:, 0:EMBED_DIM] = jnp.transpose(in_ref[...], (1, 0))


def _sc_pool_body(x_hbm, tab_hbm, out_hbm, idx_v, b0, b1, b2, b3, out_v,
                  s0, s1, s2, s3):
    wid = lax.axis_index("s") * _NC + lax.axis_index("c")
    n_idx = _PAIRS_PER_W * _IDX_PITCH

    pltpu.sync_copy(x_hbm.at[pl.ds(wid * n_idx, n_idx)], idx_v)

    bufs = (b0, b1, b2, b3)
    sems = (s0, s1, s2, s3)

    def start(j, b):
        pltpu.async_copy(
            tab_hbm.at[idx_v.at[pl.ds(j * _IDX_PITCH, _IDX_PER_GATHER)]],
            bufs[b], sems[b])

    def wait(j, b):
        pltpu.make_async_copy(
            tab_hbm.at[idx_v.at[pl.ds(j * _IDX_PITCH, _IDX_PER_GATHER)]],
            bufs[b], sems[b]).wait()

    for j in range(_NBUF - 1):
        start(j, j)

    _U = 5

    def accumulate(buf, j):
        def rbody(t, acc):
            acc = list(acc)
            for u in range(_U):
                r = t * _U + u
                for half in range(2):
                    for q in range(_QV):
                        acc[half * _QV + q] = acc[half * _QV + q] + buf[
                            half * SEQLEN + r, pl.ds(q * 16, 16)]
            return tuple(acc)

        zeros = tuple(jnp.zeros((16,), jnp.float32) for _ in range(2 * _QV))
        acc = lax.fori_loop(0, SEQLEN // _U, rbody, zeros)
        scale = jnp.float32(1.0 / SEQLEN)
        for half in range(2):
            for q in range(_QV):
                out_v[pl.ds((2 * j + half) * EMBED_DIM + q * 16, 16)] = (
                    acc[half * _QV + q] * scale)

    @pl.loop(0, _PAIRS_PER_W, step=_NBUF)
    def _(g):
        for b in range(_NBUF):
            j = g + b
            nxt = (b + _NBUF - 1) % _NBUF

            @pl.when(j + _NBUF - 1 < _PAIRS_PER_W)
            def _():
                start(j + _NBUF - 1, nxt)

            wait(j, b)
            accumulate(bufs[b], j)

    n_out = _ROWS_PER_W * EMBED_DIM
    pltpu.sync_copy(out_v, out_hbm.at[pl.ds(wid * n_out, n_out)])


def _make_sc_pool():
    mesh = plsc.VectorSubcoreMesh(core_axis_name="c", subcore_axis_name="s")
    return pl.kernel(
        _sc_pool_body,
        out_type=jax.ShapeDtypeStruct((BATCH * EMBED_DIM,), jnp.float32),
        mesh=mesh,
        compiler_params=pltpu.CompilerParams(use_tc_tiling_on_sc=True),
        scratch_types=[
            pltpu.VMEM((_PAIRS_PER_W * _IDX_PITCH,), jnp.int32),
            pltpu.VMEM((_IDX_PER_GATHER, 128), jnp.float32),
            pltpu.VMEM((_IDX_PER_GATHER, 128), jnp.float32),
            pltpu.VMEM((_IDX_PER_GATHER, 128), jnp.float32),
            pltpu.VMEM((_IDX_PER_GATHER, 128), jnp.float32),
            pltpu.VMEM((_ROWS_PER_W * EMBED_DIM,), jnp.float32),
            pltpu.SemaphoreType.DMA,
            pltpu.SemaphoreType.DMA,
            pltpu.SemaphoreType.DMA,
            pltpu.SemaphoreType.DMA,
        ],
    )


_BM = 1024


def _tc_head_body(x_ref, w_ref, b_ref, o_ref):
    logits = (
        lax.dot_general(
            w_ref[...], x_ref[...], (((1,), (1,)), ((), ())),
            preferred_element_type=jnp.float32,
        )
        + b_ref[...]
    )
    m = jnp.max(logits, axis=0, keepdims=True)
    e = jnp.exp(logits - m)
    o_ref[...] = e / jnp.sum(e, axis=0, keepdims=True)


@functools.partial(jax.jit, static_argnames=())
def _run(x, emb_table, W, b):
    xg = jnp.reshape(x.astype(jnp.int32), (BATCH // 2, _IDX_PER_GATHER))
    xg = jnp.pad(xg, ((0, 0), (0, _IDX_PITCH - _IDX_PER_GATHER)))
    x_flat = jnp.reshape(xg, (-1,))

    tab = pl.pallas_call(
        _tc_repack_body,
        grid=(_N_CBLK,),
        in_specs=[pl.BlockSpec((EMBED_DIM, _BC), lambda i: (0, i))],
        out_specs=pl.BlockSpec((_BC, 128), lambda i: (i, 0)),
        out_shape=jax.ShapeDtypeStruct((_TAB_ROWS, 128), jnp.float32),
    )(emb_table.T)

    pooled = jnp.reshape(
        _make_sc_pool()(x_flat, tab), (BATCH, EMBED_DIM))

    out_t = pl.pallas_call(
        _tc_head_body,
        grid=(BATCH // _BM,),
        in_specs=[
            pl.BlockSpec((_BM, EMBED_DIM), lambda i: (i, 0)),
            pl.BlockSpec((NUM_CLASSES, EMBED_DIM), lambda i: (0, 0)),
            pl.BlockSpec((NUM_CLASSES, 1), lambda i: (0, 0)),
        ],
        out_specs=pl.BlockSpec((NUM_CLASSES, _BM), lambda i: (0, i)),
        out_shape=jax.ShapeDtypeStruct((NUM_CLASSES, BATCH), jnp.float32),
    )(pooled, W, jnp.reshape(b, (NUM_CLASSES, 1)))
    return jnp.transpose(out_t)


def kernel(x, emb_table, W, b):
    return _run(x, emb_table, W, b)
